# Initial kernel scaffold; baseline (speedup 1.0000x reference)
import functools

import jax
import jax.numpy as jnp
from jax import lax
from jax.experimental import pallas as pl
from jax.experimental.pallas import tpu as pltpu
from jax.experimental.pallas import tpu_sc as plsc

NC = 2
NS = 16
NW = NC * NS
CHUNK = 128
SUPER = 512
GPS = SUPER // CHUNK


@functools.lru_cache(maxsize=None)
def _build(B, NNEG, D):
    dom_rows = B // NW
    neg_rows = B * NNEG // NW
    assert dom_rows % SUPER == 0
    assert neg_rows % SUPER == 0
    dom_supers = dom_rows // SUPER
    neg_supers = neg_rows // SUPER
    dom_irows = dom_rows // CHUNK
    neg_irows = neg_rows // CHUNK

    mesh = plsc.VectorSubcoreMesh(core_axis_name="c", subcore_axis_name="s")

    def body(dom_i_hbm, cod_i_hbm, neg_i_hbm, in_tab, out_tab,
             out_dom, out_cod, out_neg,
             dom_idx, cod_idx, neg_idx, buf0, buf1, g0, g1, w0, w1):
        wid = lax.axis_index("s") * NC + lax.axis_index("c")

        pltpu.sync_copy(dom_i_hbm.at[pl.ds(wid * dom_irows, dom_irows)], dom_idx)
        pltpu.sync_copy(cod_i_hbm.at[pl.ds(wid * dom_irows, dom_irows)], cod_idx)
        pltpu.sync_copy(neg_i_hbm.at[pl.ds(wid * neg_irows, neg_irows)], neg_idx)

        supers = []
        for j in range(dom_supers):
            supers.append((in_tab, dom_idx, j * GPS, out_dom,
                           wid * dom_rows + j * SUPER))
            supers.append((out_tab, cod_idx, j * GPS, out_cod,
                           wid * dom_rows + j * SUPER))
        for j in range(neg_supers):
            supers.append((out_tab, neg_idx, j * GPS, out_neg,
                           wid * neg_rows + j * SUPER))

        bufs = (buf0, buf1)
        gsems = (g0, g1)
        wsems = (w0, w1)
        n = len(supers)
        gh = [None] * n
        wh = [None] * n

        def issue_gathers(i):
            tab, idxr, ib, _, _ = supers[i]
            b = i % 2
            hs = []
            for r in range(GPS):
                hs.append(pltpu.async_copy(
                    tab.at[idxr.at[ib + r]],
                    bufs[b].at[pl.ds(r * CHUNK, CHUNK)],
                    gsems[b]))
            gh[i] = hs

        issue_gathers(0)
        for i in range(n):
            b = i % 2
            if i + 1 < n:
                if i - 1 >= 0:
                    wh[i - 1].wait()
                issue_gathers(i + 1)
            for h in gh[i]:
                h.wait()
            _, _, _, outr, ob = supers[i]
            wh[i] = pltpu.async_copy(bufs[b], outr.at[pl.ds(ob, SUPER)],
                                     wsems[b])
        wh[n - 2].wait()
        wh[n - 1].wait()

    kfn = pl.kernel(
        body,
        out_type=[
            jax.ShapeDtypeStruct((B, D), jnp.float32),
            jax.ShapeDtypeStruct((B, D), jnp.float32),
            jax.ShapeDtypeStruct((B * NNEG, D), jnp.float32),
        ],
        mesh=mesh,
        scratch_types=[
            pltpu.VMEM((dom_irows, CHUNK), jnp.int32),
            pltpu.VMEM((dom_irows, CHUNK), jnp.int32),
            pltpu.VMEM((neg_irows, CHUNK), jnp.int32),
            pltpu.VMEM((SUPER, D), jnp.float32),
            pltpu.VMEM((SUPER, D), jnp.float32),
            pltpu.SemaphoreType.DMA,
            pltpu.SemaphoreType.DMA,
            pltpu.SemaphoreType.DMA,
            pltpu.SemaphoreType.DMA,
        ],
    )
    return kfn


def kernel(domains, codomains, neg_codomains, in_embed, out_embed):
    B = domains.shape[0]
    NNEG = neg_codomains.shape[1]
    D = in_embed.shape[1]
    kfn = _build(B, NNEG, D)
    dom2d = domains.reshape(-1, CHUNK)
    cod2d = codomains.reshape(-1, CHUNK)
    neg2d = neg_codomains.reshape(-1, CHUNK)
    out_dom, out_cod, out_neg = kfn(dom2d, cod2d, neg2d, in_embed, out_embed)
    return out_dom, out_cod, out_neg.reshape(B, NNEG, D)


# speedup vs baseline: 1.0771x; 1.0771x over previous
import functools

import jax
import jax.numpy as jnp
from jax import lax
from jax.experimental import pallas as pl
from jax.experimental.pallas import tpu as pltpu
from jax.experimental.pallas import tpu_sc as plsc

NC = 2
NS = 16
NW = NC * NS
CHUNK = 128
SUPER = 512
GPS = SUPER // CHUNK


@functools.lru_cache(maxsize=None)
def _build(B, NNEG, D):
    dom_rows = B // NW
    neg_rows = B * NNEG // NW
    assert dom_rows % SUPER == 0
    assert neg_rows % SUPER == 0
    dom_supers = dom_rows // SUPER
    neg_supers = neg_rows // SUPER
    dom_irows = dom_rows // CHUNK
    neg_irows = neg_rows // CHUNK

    mesh = plsc.VectorSubcoreMesh(core_axis_name="c", subcore_axis_name="s")

    def body(dom_i_hbm, cod_i_hbm, neg_i_hbm, in_tab, out_tab,
             out_dom, out_cod, out_neg,
             dom_idx, cod_idx, neg_idx, buf0, buf1, g0, g1, w0, w1):
        wid = lax.axis_index("s") * NC + lax.axis_index("c")

        pltpu.sync_copy(dom_i_hbm.at[pl.ds(wid * dom_rows, dom_rows)], dom_idx)
        pltpu.sync_copy(cod_i_hbm.at[pl.ds(wid * dom_rows, dom_rows)], cod_idx)
        pltpu.sync_copy(neg_i_hbm.at[pl.ds(wid * neg_rows, neg_rows)], neg_idx)

        supers = []
        for j in range(dom_supers):
            supers.append((in_tab, dom_idx, j * GPS, out_dom,
                           wid * dom_rows + j * SUPER))
            supers.append((out_tab, cod_idx, j * GPS, out_cod,
                           wid * dom_rows + j * SUPER))
        for j in range(neg_supers):
            supers.append((out_tab, neg_idx, j * GPS, out_neg,
                           wid * neg_rows + j * SUPER))

        bufs = (buf0, buf1)
        gsems = (g0, g1)
        wsems = (w0, w1)
        n = len(supers)
        gh = [None] * n
        wh = [None] * n

        def issue_gathers(i):
            tab, idxr, ib, _, _ = supers[i]
            b = i % 2
            hs = []
            for r in range(GPS):
                hs.append(pltpu.async_copy(
                    tab.at[idxr.at[pl.ds((ib + r) * CHUNK, CHUNK)]],
                    bufs[b].at[pl.ds(r * CHUNK, CHUNK)],
                    gsems[b]))
            gh[i] = hs

        issue_gathers(0)
        for i in range(n):
            b = i % 2
            if i + 1 < n:
                if i - 1 >= 0:
                    wh[i - 1].wait()
                issue_gathers(i + 1)
            for h in gh[i]:
                h.wait()
            _, _, _, outr, ob = supers[i]
            wh[i] = pltpu.async_copy(bufs[b], outr.at[pl.ds(ob, SUPER)],
                                     wsems[b])
        wh[n - 2].wait()
        wh[n - 1].wait()

    kfn = pl.kernel(
        body,
        out_type=[
            jax.ShapeDtypeStruct((B, D), jnp.float32),
            jax.ShapeDtypeStruct((B, D), jnp.float32),
            jax.ShapeDtypeStruct((B * NNEG, D), jnp.float32),
        ],
        mesh=mesh,
        compiler_params=pltpu.CompilerParams(use_tc_tiling_on_sc=False),
        scratch_types=[
            pltpu.VMEM((dom_rows,), jnp.int32),
            pltpu.VMEM((dom_rows,), jnp.int32),
            pltpu.VMEM((neg_rows,), jnp.int32),
            pltpu.VMEM((SUPER, D), jnp.float32),
            pltpu.VMEM((SUPER, D), jnp.float32),
            pltpu.SemaphoreType.DMA,
            pltpu.SemaphoreType.DMA,
            pltpu.SemaphoreType.DMA,
            pltpu.SemaphoreType.DMA,
        ],
    )
    return kfn


def kernel(domains, codomains, neg_codomains, in_embed, out_embed):
    B = domains.shape[0]
    NNEG = neg_codomains.shape[1]
    D = in_embed.shape[1]
    kfn = _build(B, NNEG, D)
    neg_flat = neg_codomains.reshape(-1)
    out_dom, out_cod, out_neg = kfn(domains, codomains, neg_flat,
                                    in_embed, out_embed)
    return out_dom, out_cod, out_neg.reshape(B, NNEG, D)


# speedup vs baseline: 1.0778x; 1.0007x over previous
import functools

import jax
import jax.numpy as jnp
from jax import lax
from jax.experimental import pallas as pl
from jax.experimental.pallas import tpu as pltpu
from jax.experimental.pallas import tpu_sc as plsc

NC = 2
NS = 16
NW = NC * NS
CHUNK = 128
SUPER = 512
GPS = SUPER // CHUNK
NBUF = 3


@functools.lru_cache(maxsize=None)
def _build(B, NNEG, D):
    dom_rows = B // NW
    neg_rows = B * NNEG // NW
    assert dom_rows % SUPER == 0
    assert neg_rows % SUPER == 0
    dom_supers = dom_rows // SUPER
    neg_supers = neg_rows // SUPER
    dom_irows = dom_rows // CHUNK
    neg_irows = neg_rows // CHUNK

    mesh = plsc.VectorSubcoreMesh(core_axis_name="c", subcore_axis_name="s")

    def body(dom_i_hbm, cod_i_hbm, neg_i_hbm, in_tab, out_tab,
             out_dom, out_cod, out_neg,
             dom_idx, cod_idx, neg_idx, *bufs_and_sems):
        bufs = bufs_and_sems[:NBUF]
        gsems = bufs_and_sems[NBUF:2 * NBUF]
        wsems = bufs_and_sems[2 * NBUF:]
        wid = lax.axis_index("s") * NC + lax.axis_index("c")

        pltpu.sync_copy(dom_i_hbm.at[pl.ds(wid * dom_rows, dom_rows)], dom_idx)
        pltpu.sync_copy(cod_i_hbm.at[pl.ds(wid * dom_rows, dom_rows)], cod_idx)
        pltpu.sync_copy(neg_i_hbm.at[pl.ds(wid * neg_rows, neg_rows)], neg_idx)

        supers = []
        for j in range(dom_supers):
            supers.append((in_tab, dom_idx, j * GPS, out_dom,
                           wid * dom_rows + j * SUPER))
            supers.append((out_tab, cod_idx, j * GPS, out_cod,
                           wid * dom_rows + j * SUPER))
        for j in range(neg_supers):
            supers.append((out_tab, neg_idx, j * GPS, out_neg,
                           wid * neg_rows + j * SUPER))

        n = len(supers)
        gh = [None] * n
        wh = [None] * n

        def issue_gathers(i):
            tab, idxr, ib, _, _ = supers[i]
            b = i % NBUF
            hs = []
            for r in range(GPS):
                hs.append(pltpu.async_copy(
                    tab.at[idxr.at[pl.ds((ib + r) * CHUNK, CHUNK)]],
                    bufs[b].at[pl.ds(r * CHUNK, CHUNK)],
                    gsems[b]))
            gh[i] = hs

        for i in range(min(NBUF - 1, n)):
            issue_gathers(i)
        for i in range(n):
            b = i % NBUF
            jn = i + NBUF - 1
            if jn < n:
                if i - 1 >= 0:
                    wh[i - 1].wait()
                issue_gathers(jn)
            for h in gh[i]:
                h.wait()
            _, _, _, outr, ob = supers[i]
            wh[i] = pltpu.async_copy(bufs[b], outr.at[pl.ds(ob, SUPER)],
                                     wsems[b])
        for i in range(max(0, n - NBUF + 1), n):
            wh[i].wait()

    kfn = pl.kernel(
        body,
        out_type=[
            jax.ShapeDtypeStruct((B, D), jnp.float32),
            jax.ShapeDtypeStruct((B, D), jnp.float32),
            jax.ShapeDtypeStruct((B * NNEG, D), jnp.float32),
        ],
        mesh=mesh,
        compiler_params=pltpu.CompilerParams(use_tc_tiling_on_sc=False),
        scratch_types=[
            pltpu.VMEM((dom_rows,), jnp.int32),
            pltpu.VMEM((dom_rows,), jnp.int32),
            pltpu.VMEM((neg_rows,), jnp.int32),
        ] + [pltpu.VMEM((SUPER, D), jnp.float32) for _ in range(NBUF)]
          + [pltpu.SemaphoreType.DMA for _ in range(2 * NBUF)],
    )
    return kfn


def kernel(domains, codomains, neg_codomains, in_embed, out_embed):
    B = domains.shape[0]
    NNEG = neg_codomains.shape[1]
    D = in_embed.shape[1]
    kfn = _build(B, NNEG, D)
    neg_flat = neg_codomains.reshape(-1)
    out_dom, out_cod, out_neg = kfn(domains, codomains, neg_flat,
                                    in_embed, out_embed)
    return out_dom, out_cod, out_neg.reshape(B, NNEG, D)
